# baseline (speedup 1.0000x reference)
import jax
import jax.numpy as jnp
from jax.experimental import pallas as pl
from jax.experimental.pallas import tpu as pltpu

_NEG = -1e30
_NB = 8
_K = 256
_LANES = 128
_ROWS = 160
_PPAD = _ROWS * _LANES


def _select_kernel(s_ref, pb_ref, feat_ref, out_f_ref, out_m_ref, out_g_ref,
                   work_ref, gtt_ref):
    scene = jax.lax.broadcasted_iota(jnp.int32, (_ROWS, _NB, _LANES), 1)
    work = jnp.where(pb_ref[...] == scene, s_ref[...], _NEG)
    work_ref[...] = work
    cache0 = jnp.max(work, axis=2).T

    lane = jax.lax.broadcasted_iota(jnp.int32, (1, _LANES), 1)
    col = jax.lax.broadcasted_iota(jnp.int32, (_NB, _ROWS), 1)
    row = jax.lax.broadcasted_iota(jnp.int32, (_NB, _ROWS), 0)
    sl8 = jax.lax.broadcasted_iota(jnp.int32, (_NB, 1), 0)
    rk = jax.lax.broadcasted_iota(jnp.int32, (1, _K), 1)

    def body(r, carry):
        cache, vals, gidx = carry
        m = jnp.max(cache, axis=1, keepdims=True)
        tid = jnp.min(jnp.where(cache == m, col, _ROWS), axis=1, keepdims=True)
        for b in range(_NB):
            t_b = jnp.min(jnp.where(sl8 == b, tid, _ROWS))
            m_b = jnp.max(jnp.where(sl8 == b, m, _NEG))
            w = work_ref[pl.ds(t_b, 1), b, :]
            c_b = jnp.min(jnp.where(w == m_b, lane, _LANES))
            w2 = jnp.where(lane == c_b, _NEG, w)
            work_ref[pl.ds(t_b, 1), b, :] = w2
            cache = jnp.where((row == b) & (col == t_b), jnp.max(w2), cache)
            upd = (sl8 == b) & (rk == r)
            vals = jnp.where(upd, m_b, vals)
            gidx = jnp.where(upd, t_b * _LANES + c_b, gidx)
        return cache, vals, gidx

    _, vals, gidx = jax.lax.fori_loop(
        0, _K, body,
        (cache0,
         jnp.full((_NB, _K), _NEG, jnp.float32),
         jnp.zeros((_NB, _K), jnp.int32)))

    maskv = vals > _NEG * 0.5
    out_m_ref[...] = maskv.astype(jnp.float32)
    safe_t = jnp.where(maskv, vals, 0.0).T
    gidx_t = gidx.T

    for b in range(_NB):
        gcol = gidx_t[:, b:b + 1]

        def gat(t, acc):
            hit = (gcol == t * _LANES + lane).astype(jnp.float32)
            blk = feat_ref[pl.ds(t * _LANES, _LANES), :]
            return acc + jnp.dot(hit, blk, preferred_element_type=jnp.float32)

        acc = jax.lax.fori_loop(0, _ROWS, gat,
                                jnp.zeros((_K, 32), jnp.float32))
        out_f_ref[b, :, :] = acc[:, :16] * safe_t[:, b:b + 1]
        gtt_ref[:, b:b + 1] = jnp.where(
            maskv.T[:, b:b + 1], acc[:, 16:17], -1.0)
    out_g_ref[...] = gtt_ref[...].T


def kernel(scores, score_feats, proposal_batch, gt_instance_idxs):
    p = scores.shape[0]
    s = jnp.full((_PPAD,), _NEG, jnp.float32).at[:p].set(scores[:, 0])
    pb = jnp.full((_PPAD,), -1, jnp.int32).at[:p].set(proposal_batch)
    feat = jnp.zeros((_PPAD, 32), jnp.float32)
    feat = feat.at[:p, :16].set(score_feats)
    feat = feat.at[:p, 16].set(gt_instance_idxs.astype(jnp.float32))

    out_f, out_m, out_g = pl.pallas_call(
        _select_kernel,
        out_shape=[
            jax.ShapeDtypeStruct((_NB, _K, 16), jnp.float32),
            jax.ShapeDtypeStruct((_NB, _K), jnp.float32),
            jax.ShapeDtypeStruct((_NB, _K), jnp.float32),
        ],
        scratch_shapes=[
            pltpu.VMEM((_ROWS, _NB, _LANES), jnp.float32),
            pltpu.VMEM((_K, _NB), jnp.float32),
        ],
    )(s.reshape(_ROWS, 1, _LANES), pb.reshape(_ROWS, 1, _LANES), feat)
    return out_f, out_m, out_g


# speedup vs baseline: 2.0588x; 2.0588x over previous
import jax
import jax.numpy as jnp
from jax.experimental import pallas as pl
from jax.experimental.pallas import tpu as pltpu

_NEG = -1e30
_NB = 8
_K = 256
_LANES = 128
_ROWS = 160
_PPAD = _ROWS * _LANES


def _select_kernel(s_ref, pb_ref, feat_ref, out_f_ref, out_m_ref, out_g_ref,
                   gtt_ref):
    scene = jax.lax.broadcasted_iota(jnp.int32, (_ROWS, _NB, _LANES), 1)
    work0 = jnp.where(pb_ref[...] == scene, s_ref[...], _NEG)
    gid3 = (jax.lax.broadcasted_iota(jnp.int32, (_ROWS, _NB, _LANES), 0)
            * _LANES
            + jax.lax.broadcasted_iota(jnp.int32, (_ROWS, _NB, _LANES), 2))

    lane = jax.lax.broadcasted_iota(jnp.int32, (1, _LANES), 1)
    rk = jax.lax.broadcasted_iota(jnp.int32, (1, _K), 1)

    def body(r, carry):
        w, vals, gidx = carry
        m = jnp.max(jnp.max(w, axis=0), axis=1, keepdims=True)
        i = jnp.min(jnp.min(jnp.where(w == m[None], gid3, _PPAD),
                            axis=0), axis=1, keepdims=True)
        w = jnp.where(gid3 == i[None], _NEG, w)
        upd = rk == r
        vals = jnp.where(upd, m, vals)
        gidx = jnp.where(upd, i, gidx)
        return w, vals, gidx

    _, vals, gidx = jax.lax.fori_loop(
        0, _K, body,
        (work0,
         jnp.full((_NB, _K), _NEG, jnp.float32),
         jnp.zeros((_NB, _K), jnp.int32)))

    maskv = vals > _NEG * 0.5
    out_m_ref[...] = maskv.astype(jnp.float32)
    safe_t = jnp.where(maskv, vals, 0.0).T
    gidx_t = gidx.T

    for b in range(_NB):
        gcol = gidx_t[:, b:b + 1]

        def gat(t, acc):
            hit = (gcol == t * _LANES + lane).astype(jnp.float32)
            blk = feat_ref[pl.ds(t * _LANES, _LANES), :]
            return acc + jnp.dot(hit, blk, preferred_element_type=jnp.float32)

        acc = jax.lax.fori_loop(0, _ROWS, gat,
                                jnp.zeros((_K, 32), jnp.float32))
        out_f_ref[b, :, :] = acc[:, :16] * safe_t[:, b:b + 1]
        gtt_ref[:, b:b + 1] = jnp.where(
            maskv.T[:, b:b + 1], acc[:, 16:17], -1.0)
    out_g_ref[...] = gtt_ref[...].T


def kernel(scores, score_feats, proposal_batch, gt_instance_idxs):
    p = scores.shape[0]
    s = jnp.full((_PPAD,), _NEG, jnp.float32).at[:p].set(scores[:, 0])
    pb = jnp.full((_PPAD,), -1, jnp.int32).at[:p].set(proposal_batch)
    feat = jnp.zeros((_PPAD, 32), jnp.float32)
    feat = feat.at[:p, :16].set(score_feats)
    feat = feat.at[:p, 16].set(gt_instance_idxs.astype(jnp.float32))

    out_f, out_m, out_g = pl.pallas_call(
        _select_kernel,
        out_shape=[
            jax.ShapeDtypeStruct((_NB, _K, 16), jnp.float32),
            jax.ShapeDtypeStruct((_NB, _K), jnp.float32),
            jax.ShapeDtypeStruct((_NB, _K), jnp.float32),
        ],
        scratch_shapes=[
            pltpu.VMEM((_K, _NB), jnp.float32),
        ],
    )(s.reshape(_ROWS, 1, _LANES), pb.reshape(_ROWS, 1, _LANES), feat)
    return out_f, out_m, out_g


# speedup vs baseline: 3.2681x; 1.5874x over previous
import jax
import jax.numpy as jnp
from jax.experimental import pallas as pl
from jax.experimental.pallas import tpu as pltpu

_NEG = -1e30
_NB = 8
_K = 256
_LANES = 128
_ROWS = 160
_PPAD = _ROWS * _LANES


def _select_kernel(s_ref, pb_ref, feat_ref, out_f_ref, out_m_ref, out_g_ref,
                   gtt_ref):
    scene = jax.lax.broadcasted_iota(jnp.int32, (_ROWS, _NB, _LANES), 1)
    work0 = jnp.where(pb_ref[...] == scene, s_ref[...], _NEG)
    gid3 = (jax.lax.broadcasted_iota(jnp.int32, (_ROWS, _NB, _LANES), 0)
            * _LANES
            + jax.lax.broadcasted_iota(jnp.int32, (_ROWS, _NB, _LANES), 2))

    lane = jax.lax.broadcasted_iota(jnp.int32, (1, _LANES), 1)
    rk = jax.lax.broadcasted_iota(jnp.int32, (1, _K), 1)

    def body(r, carry):
        w, vals, gidx = carry
        m = jnp.max(jnp.max(w, axis=0), axis=1, keepdims=True)
        i = jnp.min(jnp.min(jnp.where(w == m[None], gid3, _PPAD),
                            axis=0), axis=1, keepdims=True)
        w = jnp.where(gid3 == i[None], _NEG, w)
        upd = rk == r
        vals = jnp.where(upd, m, vals)
        gidx = jnp.where(upd, i, gidx)
        return w, vals, gidx

    _, vals, gidx = jax.lax.fori_loop(
        0, _K, body,
        (work0,
         jnp.full((_NB, _K), _NEG, jnp.float32),
         jnp.zeros((_NB, _K), jnp.int32)))

    maskv = vals > _NEG * 0.5
    out_m_ref[...] = maskv.astype(jnp.float32)
    safe_t = jnp.where(maskv, vals, 0.0).T
    gidx_t = gidx.T

    lane512 = jax.lax.broadcasted_iota(jnp.int32, (1, 512), 1)
    for b in range(_NB):
        gcol = gidx_t[:, b:b + 1]

        def gat(t, acc):
            hit = (gcol == t * 512 + lane512).astype(jnp.float32)
            blk = feat_ref[pl.ds(t * 512, 512), :]
            return acc + jnp.dot(hit, blk, preferred_element_type=jnp.float32)

        acc = jax.lax.fori_loop(0, _PPAD // 512, gat,
                                jnp.zeros((_K, 32), jnp.float32))
        out_f_ref[b, :, :] = acc[:, :16] * safe_t[:, b:b + 1]
        gtt_ref[:, b:b + 1] = jnp.where(
            maskv.T[:, b:b + 1], acc[:, 16:17], -1.0)
    out_g_ref[...] = gtt_ref[...].T


def kernel(scores, score_feats, proposal_batch, gt_instance_idxs):
    p = scores.shape[0]
    s = jnp.full((_PPAD,), _NEG, jnp.float32).at[:p].set(scores[:, 0])
    pb = jnp.full((_PPAD,), -1, jnp.int32).at[:p].set(proposal_batch)
    feat = jnp.zeros((_PPAD, 32), jnp.float32)
    feat = feat.at[:p, :16].set(score_feats)
    feat = feat.at[:p, 16].set(gt_instance_idxs.astype(jnp.float32))

    out_f, out_m, out_g = pl.pallas_call(
        _select_kernel,
        out_shape=[
            jax.ShapeDtypeStruct((_NB, _K, 16), jnp.float32),
            jax.ShapeDtypeStruct((_NB, _K), jnp.float32),
            jax.ShapeDtypeStruct((_NB, _K), jnp.float32),
        ],
        scratch_shapes=[
            pltpu.VMEM((_K, _NB), jnp.float32),
        ],
    )(s.reshape(_ROWS, 1, _LANES), pb.reshape(_ROWS, 1, _LANES), feat)
    return out_f, out_m, out_g


# speedup vs baseline: 3.6047x; 1.1030x over previous
import jax
import jax.numpy as jnp
from jax.experimental import pallas as pl
from jax.experimental.pallas import tpu as pltpu

_NEG = -1e30
_NB = 8
_K = 256
_LANES = 128
_ROWS = 160
_PPAD = _ROWS * _LANES


def _select_kernel(s_ref, pb_ref, feat_ref, out_f_ref, out_m_ref, out_g_ref,
                   gtt_ref):
    scene = jax.lax.broadcasted_iota(jnp.int32, (_ROWS, _NB, _LANES), 1)
    work0 = jnp.where(pb_ref[...] == scene, s_ref[...], _NEG)
    gid3 = (jax.lax.broadcasted_iota(jnp.int32, (_ROWS, _NB, _LANES), 0)
            * _LANES
            + jax.lax.broadcasted_iota(jnp.int32, (_ROWS, _NB, _LANES), 2))

    lane = jax.lax.broadcasted_iota(jnp.int32, (1, _LANES), 1)
    rk = jax.lax.broadcasted_iota(jnp.int32, (1, _K), 1)

    def body(r, carry):
        w, vals, gidx = carry
        m = jnp.max(jnp.max(w, axis=0), axis=1, keepdims=True)
        i = jnp.min(jnp.min(jnp.where(w == m[None], gid3, _PPAD),
                            axis=0), axis=1, keepdims=True)
        w = jnp.where(gid3 == i[None], _NEG, w)
        upd = rk == r
        vals = jnp.where(upd, m, vals)
        gidx = jnp.where(upd, i, gidx)
        return w, vals, gidx

    _, vals, gidx = jax.lax.fori_loop(
        0, _K, body,
        (work0,
         jnp.full((_NB, _K), _NEG, jnp.float32),
         jnp.zeros((_NB, _K), jnp.int32)))

    maskv = vals > _NEG * 0.5
    out_m_ref[...] = maskv.astype(jnp.float32)
    safe_t = jnp.where(maskv, vals, 0.0).T
    gidx_t = gidx.T

    lane512 = jax.lax.broadcasted_iota(jnp.int32, (1, 1024), 1)
    for b in range(_NB):
        gcol = gidx_t[:, b:b + 1]

        def gat(t, acc):
            hit = (gcol == t * 1024 + lane512).astype(jnp.float32)
            blk = feat_ref[pl.ds(t * 1024, 1024), :]
            return acc + jnp.dot(hit, blk, preferred_element_type=jnp.float32)

        acc = jax.lax.fori_loop(0, _PPAD // 1024, gat,
                                jnp.zeros((_K, 32), jnp.float32))
        out_f_ref[b, :, :] = acc[:, :16] * safe_t[:, b:b + 1]
        gtt_ref[:, b:b + 1] = jnp.where(
            maskv.T[:, b:b + 1], acc[:, 16:17], -1.0)
    out_g_ref[...] = gtt_ref[...].T


def kernel(scores, score_feats, proposal_batch, gt_instance_idxs):
    p = scores.shape[0]
    s = jnp.full((_PPAD,), _NEG, jnp.float32).at[:p].set(scores[:, 0])
    pb = jnp.full((_PPAD,), -1, jnp.int32).at[:p].set(proposal_batch)
    feat = jnp.zeros((_PPAD, 32), jnp.float32)
    feat = feat.at[:p, :16].set(score_feats)
    feat = feat.at[:p, 16].set(gt_instance_idxs.astype(jnp.float32))

    out_f, out_m, out_g = pl.pallas_call(
        _select_kernel,
        out_shape=[
            jax.ShapeDtypeStruct((_NB, _K, 16), jnp.float32),
            jax.ShapeDtypeStruct((_NB, _K), jnp.float32),
            jax.ShapeDtypeStruct((_NB, _K), jnp.float32),
        ],
        scratch_shapes=[
            pltpu.VMEM((_K, _NB), jnp.float32),
        ],
    )(s.reshape(_ROWS, 1, _LANES), pb.reshape(_ROWS, 1, _LANES), feat)
    return out_f, out_m, out_g
